# baseline (speedup 1.0000x reference)
import functools

import jax
import jax.numpy as jnp
from jax import lax
from jax.experimental import pallas as pl
from jax.experimental.pallas import tpu as pltpu
from jax.experimental.pallas import tpu_sc as plsc

B = 4096
F = 26
V = 100000
D = 16
H = 128

NC = 2
NS = 16
NW = NC * NS
BPW = B // NW
RPW = BPW * F
CHUNK = 128
NCH = RPW // CHUNK


def _sc_gather_body(idx_hbm, sec_hbm, first_hbm, se_out, fo_out,
                    idx_v, rows_v, fo_v, sem_r, sem_f):
    wid = lax.axis_index("s") * NC + lax.axis_index("c")
    pltpu.sync_copy(idx_hbm.at[wid], idx_v)

    def fire(j, carry):
        pltpu.async_copy(sec_hbm.at[idx_v.at[j]],
                         rows_v.at[pl.ds(j * CHUNK, CHUNK)], sem_r)
        pltpu.async_copy(first_hbm.at[idx_v.at[j]],
                         fo_v.at[pl.ds(j * CHUNK, CHUNK)], sem_f)
        return carry

    lax.fori_loop(0, NCH, fire, 0)
    pltpu.make_async_copy(sec_hbm.at[pl.ds(0, RPW)], rows_v, sem_r).wait()
    pltpu.make_async_copy(first_hbm.at[pl.ds(0, RPW)], fo_v, sem_f).wait()
    pltpu.sync_copy(rows_v, se_out.at[wid])
    pltpu.sync_copy(fo_v, fo_out.at[wid])


_sc_gather = pl.kernel(
    _sc_gather_body,
    out_type=[jax.ShapeDtypeStruct((NW, RPW, D), jnp.float32),
              jax.ShapeDtypeStruct((NW, RPW), jnp.float32)],
    mesh=plsc.VectorSubcoreMesh(core_axis_name="c", subcore_axis_name="s"),
    scratch_types=[pltpu.VMEM((NCH, CHUNK), jnp.int32),
                   pltpu.VMEM((RPW, D), jnp.float32),
                   pltpu.VMEM((RPW,), jnp.float32),
                   pltpu.SemaphoreType.DMA,
                   pltpu.SemaphoreType.DMA],
    compiler_params=pltpu.CompilerParams(use_tc_tiling_on_sc=False),
)


RB = 256


def _tc_fm_mlp_body(se_ref, xv_ref, fo_ref, e_ref, e2_ref,
                    w0_ref, b0_ref, w1_ref, b1_ref, bc_ref, out_ref):
    xv = xv_ref[...]
    xv_exp = jnp.dot(xv, e_ref[...],
                     preferred_element_type=jnp.float32)
    s = se_ref[...] * xv_exp
    acc = jnp.dot(s, e2_ref[...], preferred_element_type=jnp.float32)
    acc2 = jnp.dot(s * s, e2_ref[...], preferred_element_type=jnp.float32)
    z = 0.5 * (acc * acc - acc2)
    h = jnp.maximum(
        jnp.dot(z, w0_ref[...], preferred_element_type=jnp.float32)
        + b0_ref[...], 0.0)
    y = jnp.maximum(
        jnp.dot(h, w1_ref[...], preferred_element_type=jnp.float32)
        + b1_ref[...], 0.0)
    first = jnp.sum(fo_ref[...] * xv, axis=1, keepdims=True)
    out_ref[...] = first + jnp.sum(y, axis=1, keepdims=True) + bc_ref[0]


_tc_fm_mlp = pl.pallas_call(
    _tc_fm_mlp_body,
    grid=(B // RB,),
    in_specs=[
        pl.BlockSpec((RB, F * D), lambda i: (i, 0)),
        pl.BlockSpec((RB, F), lambda i: (i, 0)),
        pl.BlockSpec((RB, F), lambda i: (i, 0)),
        pl.BlockSpec((F, F * D), lambda i: (0, 0)),
        pl.BlockSpec((F * D, D), lambda i: (0, 0)),
        pl.BlockSpec((D, H), lambda i: (0, 0)),
        pl.BlockSpec((1, H), lambda i: (0, 0)),
        pl.BlockSpec((H, H), lambda i: (0, 0)),
        pl.BlockSpec((1, H), lambda i: (0, 0)),
        pl.BlockSpec(memory_space=pltpu.SMEM),
    ],
    out_specs=pl.BlockSpec((RB, 1), lambda i: (i, 0)),
    out_shape=jax.ShapeDtypeStruct((B, 1), jnp.float32),
)


def kernel(Xi, Xv, first_tables, second_tables, W0, b0, W1, b1, b_const):
    idx = Xi[:, :, 0] + (jnp.arange(F, dtype=jnp.int32) * V)[None, :]
    idx3 = idx.reshape(NW, NCH, CHUNK)
    sec_flat = second_tables.reshape(F * V, D)
    first_flat = first_tables.reshape(F * V)
    se, fo = _sc_gather(idx3, sec_flat, first_flat)
    se2 = se.reshape(B, F * D)
    fo2 = fo.reshape(B, F)
    expand = jnp.kron(jnp.eye(F, dtype=jnp.float32), jnp.ones((1, D), jnp.float32))
    reduce = jnp.kron(jnp.ones((F, 1), jnp.float32), jnp.eye(D, dtype=jnp.float32))
    out = _tc_fm_mlp(se2, Xv, fo2, expand, reduce,
                     W0, b0.reshape(1, H), W1, b1.reshape(1, H),
                     b_const.reshape(1))
    return out.reshape(B)


# speedup vs baseline: 2.3882x; 2.3882x over previous
import jax
import jax.numpy as jnp
from jax import lax
from jax.experimental import pallas as pl
from jax.experimental.pallas import tpu as pltpu
from jax.experimental.pallas import tpu_sc as plsc

B = 4096
F = 26
V = 100000
D = 16
H = 128

NC = 2
NS = 16
NW = NC * NS
BPW = B // NW
NG = BPW // 16


def _sc_body(idx_hbm, xv_hbm, sec_hbm, first_hbm, z_out, fo_out,
             idx_v, xv_v, buf, fo_v, z_v, fo_s, sem_r, sem_f):
    wid = lax.axis_index("s") * NC + lax.axis_index("c")
    base = wid * BPW
    pltpu.sync_copy(idx_hbm.at[:, pl.ds(base, BPW)], idx_v)
    pltpu.sync_copy(xv_hbm.at[:, pl.ds(base, BPW)], xv_v)

    def fire(f, carry):
        pltpu.async_copy(first_hbm.at[f].at[idx_v.at[f]], fo_v.at[f], sem_f)
        for d in range(D):
            row = f * D + d
            pltpu.async_copy(sec_hbm.at[row].at[idx_v.at[f]],
                             buf.at[row], sem_r)
        return carry

    lax.fori_loop(0, F, fire, 0)
    pltpu.make_async_copy(sec_hbm.at[pl.ds(0, F * D), pl.ds(0, BPW)],
                          buf, sem_r).wait()
    pltpu.make_async_copy(first_hbm.at[pl.ds(0, F), pl.ds(0, BPW)],
                          fo_v, sem_f).wait()

    def fm(i, carry):
        d = i // NG
        col = (i % NG) * 16
        acc = jnp.zeros((16,), jnp.float32)
        acc2 = jnp.zeros((16,), jnp.float32)
        for f in range(F):
            s = buf[f * D + d, pl.ds(col, 16)] * xv_v[f, pl.ds(col, 16)]
            acc = acc + s
            acc2 = acc2 + s * s
        z_v[d, pl.ds(col, 16)] = 0.5 * (acc * acc - acc2)
        return carry

    lax.fori_loop(0, D * NG, fm, 0)

    def first_order(g, carry):
        col = g * 16
        facc = jnp.zeros((16,), jnp.float32)
        for f in range(F):
            facc = facc + fo_v[f, pl.ds(col, 16)] * xv_v[f, pl.ds(col, 16)]
        fo_s[pl.ds(col, 16)] = facc
        return carry

    lax.fori_loop(0, NG, first_order, 0)
    pltpu.sync_copy(z_v, z_out.at[:, pl.ds(base, BPW)])
    pltpu.sync_copy(fo_s, fo_out.at[pl.ds(base, BPW)])


_sc_fm = pl.kernel(
    _sc_body,
    out_type=[jax.ShapeDtypeStruct((D, B), jnp.float32),
              jax.ShapeDtypeStruct((B,), jnp.float32)],
    mesh=plsc.VectorSubcoreMesh(core_axis_name="c", subcore_axis_name="s"),
    scratch_types=[pltpu.VMEM((F, BPW), jnp.int32),
                   pltpu.VMEM((F, BPW), jnp.float32),
                   pltpu.VMEM((F * D, BPW), jnp.float32),
                   pltpu.VMEM((F, BPW), jnp.float32),
                   pltpu.VMEM((D, BPW), jnp.float32),
                   pltpu.VMEM((BPW,), jnp.float32),
                   pltpu.SemaphoreType.DMA,
                   pltpu.SemaphoreType.DMA],
    compiler_params=pltpu.CompilerParams(use_tc_tiling_on_sc=False),
)


CB = 512


def _tc_body(zt_ref, fos_ref, w0_ref, w1_ref, b0_ref, b1_ref, bc_ref,
             out_ref):
    zt = zt_ref[...]
    h = jnp.maximum(
        lax.dot_general(w0_ref[...], zt, (((0,), (0,)), ((), ())),
                        preferred_element_type=jnp.float32)
        + b0_ref[...], 0.0)
    y = jnp.maximum(
        lax.dot_general(w1_ref[...], h, (((0,), (0,)), ((), ())),
                        preferred_element_type=jnp.float32)
        + b1_ref[...], 0.0)
    out_ref[...] = jnp.sum(y, axis=0) + fos_ref[...] + bc_ref[0]


_tc_mlp = pl.pallas_call(
    _tc_body,
    grid=(B // CB,),
    in_specs=[
        pl.BlockSpec((D, CB), lambda i: (0, i)),
        pl.BlockSpec((CB,), lambda i: (i,)),
        pl.BlockSpec((D, H), lambda i: (0, 0)),
        pl.BlockSpec((H, H), lambda i: (0, 0)),
        pl.BlockSpec((H, 1), lambda i: (0, 0)),
        pl.BlockSpec((H, 1), lambda i: (0, 0)),
        pl.BlockSpec(memory_space=pltpu.SMEM),
    ],
    out_specs=pl.BlockSpec((CB,), lambda i: (i,)),
    out_shape=jax.ShapeDtypeStruct((B,), jnp.float32),
)


def kernel(Xi, Xv, first_tables, second_tables, W0, b0, W1, b1, b_const):
    idx_t = Xi[:, :, 0].T
    xv_t = Xv.T
    sec_t = second_tables.transpose(0, 2, 1).reshape(F * D, V)
    first_t = first_tables[:, :, 0]
    z_t, fo_sum = _sc_fm(idx_t, xv_t, sec_t, first_t)
    return _tc_mlp(z_t, fo_sum, W0, W1,
                   b0.reshape(H, 1), b1.reshape(H, 1), b_const.reshape(1))


# speedup vs baseline: 3.0795x; 1.2895x over previous
import jax
import jax.numpy as jnp
from jax import lax
from jax.experimental import pallas as pl
from jax.experimental.pallas import tpu as pltpu
from jax.experimental.pallas import tpu_sc as plsc

B = 4096
F = 26
V = 100000
VP = 100096
D = 16
H = 128

NC = 2
NS = 16
NW = NC * NS
BPW = B // NW
NG = BPW // 16


def _sc_body(idx_hbm, xv_hbm, sec_hbm, first_hbm, z_out, fo_out,
             idx_v, xv_v, buf, fo_v, z_v, fo_s, sem_r, sem_f):
    wid = lax.axis_index("s") * NC + lax.axis_index("c")
    base = wid * BPW
    pltpu.sync_copy(idx_hbm.at[:, pl.ds(base, BPW)], idx_v)
    pltpu.sync_copy(xv_hbm.at[:, pl.ds(base, BPW)], xv_v)

    def fire(f, carry):
        off = pl.multiple_of(f * VP, 8)
        pltpu.async_copy(first_hbm.at[pl.ds(off, V)].at[idx_v.at[f]],
                         fo_v.at[f], sem_f)
        for d in range(D):
            row = f * D + d
            pltpu.async_copy(sec_hbm.at[row].at[idx_v.at[f]],
                             buf.at[row], sem_r)
        return carry

    lax.fori_loop(0, F, fire, 0)
    pltpu.make_async_copy(sec_hbm.at[pl.ds(0, F * D), pl.ds(0, BPW)],
                          buf, sem_r).wait()
    pltpu.make_async_copy(sec_hbm.at[pl.ds(0, F), pl.ds(0, BPW)],
                          fo_v, sem_f).wait()

    def fm(i, carry):
        d = i // NG
        col = (i % NG) * 16
        acc = jnp.zeros((16,), jnp.float32)
        acc2 = jnp.zeros((16,), jnp.float32)
        for f in range(F):
            s = buf[f * D + d, pl.ds(col, 16)] * xv_v[f, pl.ds(col, 16)]
            acc = acc + s
            acc2 = acc2 + s * s
        z_v[d, pl.ds(col, 16)] = 0.5 * (acc * acc - acc2)
        return carry

    lax.fori_loop(0, D * NG, fm, 0)

    def first_order(g, carry):
        col = g * 16
        facc = jnp.zeros((16,), jnp.float32)
        for f in range(F):
            facc = facc + fo_v[f, pl.ds(col, 16)] * xv_v[f, pl.ds(col, 16)]
        fo_s[pl.ds(col, 16)] = facc
        return carry

    lax.fori_loop(0, NG, first_order, 0)
    pltpu.sync_copy(z_v, z_out.at[:, pl.ds(base, BPW)])
    pltpu.sync_copy(fo_s, fo_out.at[pl.ds(base, BPW)])


_sc_fm = pl.kernel(
    _sc_body,
    out_type=[jax.ShapeDtypeStruct((D, B), jnp.float32),
              jax.ShapeDtypeStruct((B,), jnp.float32)],
    mesh=plsc.VectorSubcoreMesh(core_axis_name="c", subcore_axis_name="s"),
    scratch_types=[pltpu.VMEM((F, BPW), jnp.int32),
                   pltpu.VMEM((F, BPW), jnp.float32),
                   pltpu.VMEM((F * D, BPW), jnp.float32),
                   pltpu.VMEM((F, BPW), jnp.float32),
                   pltpu.VMEM((D, BPW), jnp.float32),
                   pltpu.VMEM((BPW,), jnp.float32),
                   pltpu.SemaphoreType.DMA,
                   pltpu.SemaphoreType.DMA],
    compiler_params=pltpu.CompilerParams(use_tc_tiling_on_sc=False),
)


CB = 512


def _tc_body(zt_ref, fos_ref, w0_ref, w1_ref, b0_ref, b1_ref, bc_ref,
             out_ref):
    zt = zt_ref[...]
    h = jnp.maximum(
        lax.dot_general(w0_ref[...], zt, (((0,), (0,)), ((), ())),
                        preferred_element_type=jnp.float32)
        + b0_ref[...], 0.0)
    y = jnp.maximum(
        lax.dot_general(w1_ref[...], h, (((0,), (0,)), ((), ())),
                        preferred_element_type=jnp.float32)
        + b1_ref[...], 0.0)
    out_ref[...] = jnp.sum(y, axis=0) + fos_ref[...] + bc_ref[0]


_tc_mlp = pl.pallas_call(
    _tc_body,
    grid=(B // CB,),
    in_specs=[
        pl.BlockSpec((D, CB), lambda i: (0, i)),
        pl.BlockSpec((CB,), lambda i: (i,)),
        pl.BlockSpec((D, H), lambda i: (0, 0)),
        pl.BlockSpec((H, H), lambda i: (0, 0)),
        pl.BlockSpec((H, 1), lambda i: (0, 0)),
        pl.BlockSpec((H, 1), lambda i: (0, 0)),
        pl.BlockSpec(memory_space=pltpu.SMEM),
    ],
    out_specs=pl.BlockSpec((CB,), lambda i: (i,)),
    out_shape=jax.ShapeDtypeStruct((B,), jnp.float32),
)


def kernel(Xi, Xv, first_tables, second_tables, W0, b0, W1, b1, b_const):
    idx_t = Xi[:, :, 0].T
    xv_t = Xv.T
    sec_t = second_tables.transpose(0, 2, 1).reshape(F * D, V)
    first_lin = jnp.pad(first_tables[:, :, 0],
                        ((0, 0), (0, VP - V))).reshape(F * VP)
    z_t, fo_sum = _sc_fm(idx_t, xv_t, sec_t, first_lin)
    return _tc_mlp(z_t, fo_sum, W0, W1,
                   b0.reshape(H, 1), b1.reshape(H, 1), b_const.reshape(1))


# speedup vs baseline: 4.7405x; 1.5394x over previous
import jax
import jax.numpy as jnp
from jax import lax
from jax.experimental import pallas as pl
from jax.experimental.pallas import tpu as pltpu
from jax.experimental.pallas import tpu_sc as plsc

B = 4096
F = 26
V = 100000
VP = 100096
D = 16
H = 128

NC = 2
NS = 16
NW = NC * NS
BPW = B // NW
NG = BPW // 16


SBLK = 800768
SLEN = 799872


def _sc_body(idx_hbm, xv_hbm, sec_hbm, first_hbm, z_out, fo_out,
             idx_v, idxg_v, xv_v, buf, fo_v, z_v, fo_s, sem_r, sem_f):
    wid = lax.axis_index("s") * NC + lax.axis_index("c")
    base = wid * BPW
    pltpu.sync_copy(idx_hbm.at[:, pl.ds(base, BPW)], idx_v)
    pltpu.sync_copy(xv_hbm.at[:, pl.ds(base, BPW)], xv_v)

    def idxg(i, carry):
        f = i // NG
        col = (i % NG) * 16
        g = idx_v[f, pl.ds(col, 16)]
        idxg_v[f, pl.ds(col, 16)] = g + (g >> 7) * 896
        return carry

    lax.fori_loop(0, F * NG, idxg, 0)

    def fire(f, carry):
        off = pl.multiple_of(f * VP, 8)
        pltpu.async_copy(first_hbm.at[pl.ds(off, V)].at[idx_v.at[f]],
                         fo_v.at[pl.ds(f * BPW, BPW)], sem_f)
        for d in range(D):
            sb = pl.multiple_of(f * (2 * SBLK) + (d // 8) * SBLK
                                + (d % 8) * 128, 8)
            pltpu.async_copy(sec_hbm.at[pl.ds(sb, SLEN)].at[idxg_v.at[f]],
                             buf.at[pl.ds((f * D + d) * BPW, BPW)], sem_r)
        return carry

    lax.fori_loop(0, F, fire, 0)
    pltpu.make_async_copy(sec_hbm.at[pl.ds(0, F * D * BPW)], buf,
                          sem_r).wait()
    pltpu.make_async_copy(sec_hbm.at[pl.ds(0, F * BPW)], fo_v,
                          sem_f).wait()

    def fm(i, carry):
        d = i // NG
        col = (i % NG) * 16
        acc = jnp.zeros((16,), jnp.float32)
        acc2 = jnp.zeros((16,), jnp.float32)
        for f in range(F):
            s = (buf[pl.ds((f * D + d) * BPW + col, 16)]
                 * xv_v[f, pl.ds(col, 16)])
            acc = acc + s
            acc2 = acc2 + s * s
        z_v[d, pl.ds(col, 16)] = 0.5 * (acc * acc - acc2)
        return carry

    lax.fori_loop(0, D * NG, fm, 0)

    def first_order(g, carry):
        col = g * 16
        facc = jnp.zeros((16,), jnp.float32)
        for f in range(F):
            facc = facc + (fo_v[pl.ds(f * BPW + col, 16)]
                           * xv_v[f, pl.ds(col, 16)])
        fo_s[pl.ds(col, 16)] = facc
        return carry

    lax.fori_loop(0, NG, first_order, 0)
    pltpu.sync_copy(z_v, z_out.at[:, pl.ds(base, BPW)])
    pltpu.sync_copy(fo_s, fo_out.at[pl.ds(base, BPW)])


_sc_fm = pl.kernel(
    _sc_body,
    out_type=[jax.ShapeDtypeStruct((D, B), jnp.float32),
              jax.ShapeDtypeStruct((B,), jnp.float32)],
    mesh=plsc.VectorSubcoreMesh(core_axis_name="c", subcore_axis_name="s"),
    scratch_types=[pltpu.VMEM((F, BPW), jnp.int32),
                   pltpu.VMEM((F, BPW), jnp.int32),
                   pltpu.VMEM((F, BPW), jnp.float32),
                   pltpu.VMEM((F * D * BPW,), jnp.float32),
                   pltpu.VMEM((F * BPW,), jnp.float32),
                   pltpu.VMEM((D, BPW), jnp.float32),
                   pltpu.VMEM((BPW,), jnp.float32),
                   pltpu.SemaphoreType.DMA,
                   pltpu.SemaphoreType.DMA],
    compiler_params=pltpu.CompilerParams(use_tc_tiling_on_sc=False),
)


CB = 512


def _tc_body(zt_ref, fos_ref, w0_ref, w1_ref, b0_ref, b1_ref, bc_ref,
             out_ref):
    zt = zt_ref[...]
    h = jnp.maximum(
        lax.dot_general(w0_ref[...], zt, (((0,), (0,)), ((), ())),
                        preferred_element_type=jnp.float32)
        + b0_ref[...], 0.0)
    y = jnp.maximum(
        lax.dot_general(w1_ref[...], h, (((0,), (0,)), ((), ())),
                        preferred_element_type=jnp.float32)
        + b1_ref[...], 0.0)
    out_ref[...] = jnp.sum(y, axis=0) + fos_ref[...] + bc_ref[0]


_tc_mlp = pl.pallas_call(
    _tc_body,
    grid=(B // CB,),
    in_specs=[
        pl.BlockSpec((D, CB), lambda i: (0, i)),
        pl.BlockSpec((CB,), lambda i: (i,)),
        pl.BlockSpec((D, H), lambda i: (0, 0)),
        pl.BlockSpec((H, H), lambda i: (0, 0)),
        pl.BlockSpec((H, 1), lambda i: (0, 0)),
        pl.BlockSpec((H, 1), lambda i: (0, 0)),
        pl.BlockSpec(memory_space=pltpu.SMEM),
    ],
    out_specs=pl.BlockSpec((CB,), lambda i: (i,)),
    out_shape=jax.ShapeDtypeStruct((B,), jnp.float32),
)


def kernel(Xi, Xv, first_tables, second_tables, W0, b0, W1, b1, b_const):
    idx_t = Xi[:, :, 0].T
    xv_t = Xv.T
    nvb = VP // 128
    sec_lin = (jnp.pad(second_tables, ((0, 0), (0, VP - V), (0, 0)))
               .transpose(0, 2, 1).reshape(F, 2, 8, nvb, 128)
               .transpose(0, 1, 3, 2, 4).reshape(F * 2 * nvb * 8 * 128))
    first_lin = jnp.pad(first_tables[:, :, 0],
                        ((0, 0), (0, VP - V))).reshape(F * VP)
    z_t, fo_sum = _sc_fm(idx_t, xv_t, sec_lin, first_lin)
    return _tc_mlp(z_t, fo_sum, W0, W1,
                   b0.reshape(H, 1), b1.reshape(H, 1), b_const.reshape(1))
